# baseline (speedup 1.0000x reference)
import functools
import jax
import jax.numpy as jnp
from jax import lax
from jax.experimental import pallas as pl
from jax.experimental.pallas import tpu as pltpu
from jax.experimental.pallas import tpu_sc as plsc

_L = 16


def _make_sc_add(B, T, D):
    info = plsc.get_sparse_core_info()
    NC, NS = info.num_cores, info.num_subcores
    NW = NC * NS
    rows = B * T
    rows_per_w = rows // NW
    chunk_rows = 64
    n_chunks = rows_per_w // chunk_rows
    CH = chunk_rows * D
    n_vec = CH // _L

    mesh = plsc.VectorSubcoreMesh(core_axis_name="c", subcore_axis_name="s")

    @functools.partial(
        pl.kernel,
        mesh=mesh,
        out_type=jax.ShapeDtypeStruct((rows * D,), jnp.float32),
        scratch_types=[
            pltpu.VMEM((CH,), jnp.float32),
            pltpu.VMEM((CH,), jnp.float32),
        ],
    )
    def sc_add(x_hbm, pos_hbm, out_hbm, xv, pv):
        wid = lax.axis_index("s") * NC + lax.axis_index("c")
        row0 = wid * rows_per_w
        prow0 = lax.rem(row0, T)

        def chunk_body(c, _):
            xoff = (row0 + c * chunk_rows) * D
            poff = (prow0 + c * chunk_rows) * D
            pltpu.sync_copy(x_hbm.at[pl.ds(xoff, CH)], xv)
            pltpu.sync_copy(pos_hbm.at[pl.ds(poff, CH)], pv)

            def vec_body(i, _):
                sl = pl.ds(i * _L, _L)
                xv[sl] = xv[sl] + pv[sl]
                return ()

            lax.fori_loop(0, n_vec, vec_body, ())
            pltpu.sync_copy(xv, out_hbm.at[pl.ds(xoff, CH)])
            return ()

        lax.fori_loop(0, n_chunks, chunk_body, ())

    return sc_add


def kernel(x, pos_table):
    T, D = pos_table.shape
    xr = x.reshape(-1, T, D)
    B = xr.shape[0]
    sc_add = _make_sc_add(B, T, D)
    out = sc_add(xr.reshape(-1), pos_table.reshape(-1))
    return out.reshape(B, T, D)
